# baseline (speedup 1.0000x reference)
import functools

import jax
import jax.numpy as jnp
from jax import lax
from jax.experimental import pallas as pl
from jax.experimental.pallas import tpu as pltpu
from jax.experimental.pallas import tpu_sc as plsc

D = 64
L = 16
CHUNK = 512
NG = 2
GROWS = CHUNK // NG
NW = 32
UNROLL = 16


def _lookup(ids_flat, weight):
    b_total = ids_flat.shape[0]
    per_w = b_total // NW
    nch = per_w // CHUNK
    assert nch % 2 == 0

    mesh = plsc.VectorSubcoreMesh(core_axis_name="c", subcore_axis_name="s")

    @functools.partial(
        pl.kernel,
        out_type=jax.ShapeDtypeStruct((b_total * D,), jnp.bfloat16),
        mesh=mesh,
        scratch_types=[
            pltpu.VMEM((CHUNK,), jnp.int32),
            pltpu.VMEM((CHUNK,), jnp.int32),
            pltpu.VMEM((CHUNK, D), jnp.float32),
            pltpu.VMEM((CHUNK, D), jnp.float32),
            pltpu.VMEM((CHUNK * D,), jnp.bfloat16),
            pltpu.VMEM((CHUNK * D,), jnp.bfloat16),
            pltpu.SemaphoreType.DMA,
            pltpu.SemaphoreType.DMA,
            pltpu.SemaphoreType.DMA,
            pltpu.SemaphoreType.DMA,
        ],
        compiler_params=pltpu.CompilerParams(
            needs_layout_passes=False, use_tc_tiling_on_sc=False
        ),
    )
    def run(idx_hbm, tbl_hbm, out_hbm, idx_a, idx_b, rows_a, rows_b,
            out_a, out_b, gsem_a, gsem_b, osem_a, osem_b):
        cid = lax.axis_index("c")
        sid = lax.axis_index("s")
        wid = sid * 2 + cid
        flat0 = wid * per_w
        oflat0 = wid * per_w * D

        iota = lax.iota(jnp.int32, L)
        bufs = ((idx_a, rows_a, out_a, gsem_a, osem_a),
                (idx_b, rows_b, out_b, gsem_b, osem_b))

        def start(t, bi):
            idx_v, rows_v, _, gsem, _ = bufs[bi]
            pltpu.sync_copy(
                idx_hbm.at[pl.ds(flat0 + t * CHUNK, CHUNK)], idx_v
            )
            for g in range(NG):
                pltpu.async_copy(
                    tbl_hbm.at[idx_v.at[pl.ds(g * GROWS, GROWS)]],
                    rows_v.at[pl.ds(g * GROWS, GROWS)],
                    gsem,
                )

        def wait_gather(bi, g):
            idx_v, rows_v, _, gsem, _ = bufs[bi]
            pltpu.make_async_copy(
                tbl_hbm.at[idx_v.at[pl.ds(g * GROWS, GROWS)]],
                rows_v.at[pl.ds(g * GROWS, GROWS)],
                gsem,
            ).wait()

        def out_slice(t):
            return out_hbm.at[pl.ds(oflat0 + t * CHUNK * D, CHUNK * D)]

        def fire_out(t, bi):
            _, _, out_v, _, osem = bufs[bi]
            pltpu.async_copy(out_v, out_slice(t), osem)

        def wait_out(t, bi):
            _, _, out_v, _, osem = bufs[bi]
            pltpu.make_async_copy(out_v, out_slice(t), osem).wait()

        def cast(bi, r0, nrows):
            _, rows_v, out_v, _, _ = bufs[bi]

            def cast_body(gi, c2):
                for u in range(UNROLL):
                    j = r0 + gi * (UNROLL // 2) + u // 2
                    jv = jnp.full((L,), j, jnp.int32)
                    c0 = (u % 2) * 32
                    ev = plsc.load_gather(rows_v, [jv, c0 + 2 * iota])
                    od = plsc.load_gather(rows_v, [jv, c0 + 2 * iota + 1])
                    p = plsc.pack(ev, od, format=plsc.PackFormat.INTERLEAVED)
                    out_v[pl.ds(r0 * D + gi * (UNROLL * 32) + u * 32, 32)] = p
                return c2

            lax.fori_loop(0, nrows * D // (UNROLL * 32), cast_body, 0)

        start(0, 0)

        def body(p, carry):
            t0 = 2 * p
            t1 = 2 * p + 1
            start(t1, 1)

            @pl.when(p > 0)
            def _():
                wait_out(t0 - 2, 0)

            for g in range(NG):
                wait_gather(0, g)
                cast(0, g * GROWS, GROWS)
            fire_out(t0, 0)

            @pl.when(p < nch // 2 - 1)
            def _():
                start(t0 + 2, 0)

            @pl.when(p > 0)
            def _():
                wait_out(t1 - 2, 1)

            for g in range(NG):
                wait_gather(1, g)
                cast(1, g * GROWS, GROWS)
            fire_out(t1, 1)
            return carry

        lax.fori_loop(0, nch // 2, body, 0)
        wait_out(nch - 2, 0)
        wait_out(nch - 1, 1)

    return run(ids_flat, weight)


def kernel(input_ids, weight):
    b, s = input_ids.shape
    ids = input_ids.reshape(-1).astype(jnp.int32)
    out = _lookup(ids, weight)
    return out.reshape(b, s, D)
